# baseline (speedup 1.0000x reference)
import functools

import jax
import jax.numpy as jnp
from jax import lax
from jax.experimental import pallas as pl
from jax.experimental.pallas import tpu as pltpu
from jax.experimental.pallas import tpu_sc as plsc

N = 10000
E = 320000
D = 128
DE = 16
DEE = 32
NG = 50
NSEG = 100
NLAYERS = 4
JW = 0.5

EW = 128
NC = 2
NS = 16
NW = NC * NS
BE = 128
NB = 80
EPW = NB * BE
EPAD = NW * EPW
NPAD = 10112
ZROWS = NPAD // NS
WROWS = 632
WLAST = N - 15 * WROWS

_PREC = jax.lax.Precision.HIGHEST


def _r(t):
    return t.astype(jnp.bfloat16).astype(jnp.float32)


def _mm(a, b):
    return jax.lax.dot_general(a, b, (((a.ndim - 1,), (0,)), ((), ())),
                               precision=_PREC, preferred_element_type=jnp.float32)


def _mm_t(a, b):
    return jax.lax.dot_general(a, b, (((1,), (1,)), ((), ())),
                               precision=_PREC, preferred_element_type=jnp.float32)


def _gnorm(v, w, b, ms):
    m = jnp.mean(v, axis=0, keepdims=True)
    o = v - m * ms
    var = jnp.mean(o * o, axis=0, keepdims=True)
    return w * o / jnp.sqrt(var + 1e-5) + b


@functools.lru_cache(maxsize=None)
def _make_sc_agg(gather: bool, width: int):
    mesh = plsc.VectorSubcoreMesh(core_axis_name="c", subcore_axis_name="s",
                                  num_cores=NC, num_subcores=NS)
    out_type = jax.ShapeDtypeStruct((NC, N, width), jnp.float32)
    scratch = [
        pltpu.VMEM((NB, BE), jnp.int32),
        pltpu.VMEM((NB, BE), jnp.int32),
        pltpu.VMEM((BE, width), jnp.float32),
        pltpu.VMEM_SHARED((NPAD, width), jnp.float32),
        pltpu.SemaphoreType.DMA,
    ]

    def body(pay, srcb, dstb, z, out, srcv, dstv, buf, acc, sem):
        c = lax.axis_index("c")
        s = lax.axis_index("s")
        wid = c * NS + s

        pltpu.sync_copy(z.at[pl.ds(s * ZROWS, ZROWS)],
                        acc.at[pl.ds(s * ZROWS, ZROWS)])
        if gather:
            pltpu.sync_copy(srcb.at[wid], srcv)
        pltpu.sync_copy(dstb.at[wid], dstv)
        plsc.subcore_barrier()

        def step(j, carry):
            if gather:
                pltpu.async_copy(pay.at[srcv.at[j]], buf, sem).wait()
            else:
                pltpu.async_copy(pay.at[wid, j], buf, sem).wait()
            pltpu.sync_copy(buf, acc.at[dstv.at[j]], add=True)
            return carry

        lax.fori_loop(0, NB, step, 0)
        plsc.subcore_barrier()

        @pl.when(s < NS - 1)
        def _():
            pltpu.sync_copy(acc.at[pl.ds(s * WROWS, WROWS)],
                            out.at[c, pl.ds(s * WROWS, WROWS)])

        @pl.when(s == NS - 1)
        def _():
            pltpu.sync_copy(acc.at[pl.ds((NS - 1) * WROWS, WLAST)],
                            out.at[c, pl.ds((NS - 1) * WROWS, WLAST)])

    return pl.kernel(body, out_type=out_type, mesh=mesh, scratch_types=scratch)


def _sc_agg_x(x, srcb, dstb, zx):
    return _make_sc_agg(True, D)(x, srcb, dstb, zx)


def _sc_agg_eft(eftp, srcb, dstb, ze):
    return _make_sc_agg(False, EW)(eftp, srcb, dstb, ze)


_EBLK = 4096


def _edge_mlp_body(ef_ref, w_ref, b_ref, o_ref):
    o_ref[...] = _r(jnp.maximum(
        _mm(_r(ef_ref[...]), _r(w_ref[...])) + b_ref[...], 0.0))


def _edge_mlp(efp, wp, bp):
    return pl.pallas_call(
        _edge_mlp_body,
        grid=(EPAD // _EBLK,),
        in_specs=[
            pl.BlockSpec((_EBLK, DE), lambda i: (i, 0)),
            pl.BlockSpec((DE, EW), lambda i: (0, 0)),
            pl.BlockSpec((1, EW), lambda i: (0, 0)),
        ],
        out_specs=pl.BlockSpec((_EBLK, EW), lambda i: (i, 0)),
        out_shape=jax.ShapeDtypeStruct((EPAD, EW), jnp.float32),
    )(efp, wp, bp)


def _agg_epilogue(ax_ref, ae_ref, half, wx, we, tb, gw, gb, gm):
    agg = ax_ref[0, half] + ax_ref[1, half]
    ae = ae_ref[0, half] + ae_ref[1, half]
    cnt = ae[:, DEE:DEE + 1]
    pre = (_mm(agg, wx) + _mm(ae[:, :DEE], we) + cnt * tb) / jnp.maximum(cnt, 1.0)
    return _gnorm(jnp.maximum(pre, 0.0), gw, gb, gm)


def _layer0_body(ax_ref, ae_ref, wx_ref, we_ref, tb_ref, gw_ref, gb_ref,
                 gm_ref, o_ref, or_ref):
    wx = _r(wx_ref[...])
    we = _r(we_ref[...])
    tb = tb_ref[...]
    gw = gw_ref[...]
    gb = gb_ref[...]
    gm = gm_ref[...]
    for half in range(2):
        res = _agg_epilogue(ax_ref, ae_ref, half, wx, we, tb, gw, gb, gm)
        o_ref[half] = res
        or_ref[half] = _r(res)


def _layer_body(x0_ref, ax_ref, ae_ref, wx_ref, we_ref, tb_ref, gw_ref,
                gb_ref, gm_ref, hq_ref, hk_ref, hv_ref, gq_ref, gk_ref,
                gv_ref, l1_ref, l2_ref, lb_ref, o_ref, aux_ref, *, want_mean):
    wx = _r(wx_ref[...])
    we = _r(we_ref[...])
    tb = tb_ref[...]
    gw = gw_ref[...]
    gb = gb_ref[...]
    gm = gm_ref[...]
    xn = [_agg_epilogue(ax_ref, ae_ref, h, wx, we, tb, gw, gb, gm)
          for h in range(2)]
    x0h = x0_ref[0]
    x0g = x0_ref[1]
    x0hr = _r(x0h)
    x0gr = _r(x0g)
    hq = jnp.maximum(_mm(x0hr, _r(hq_ref[...])), 0.0)
    hk = jnp.maximum(_mm(x0hr, _r(hk_ref[...])), 0.0)
    hv = _mm(x0hr, _r(hv_ref[...]))
    gq = jnp.maximum(_mm(x0gr, _r(gq_ref[...])), 0.0)
    gk = jnp.maximum(_mm(x0gr, _r(gk_ref[...])), 0.0)
    gv = _mm(x0gr, _r(gv_ref[...]))
    ah = _mm(_r(jax.nn.softmax(_mm_t(_r(hq), _r(gk)), axis=1)), _r(gv))
    ag = _mm(_r(jax.nn.softmax(_mm_t(_r(gq), _r(hk)), axis=1)), _r(hv))
    xc = [_gnorm(ah, gw, gb, gm), _gnorm(ag, gw, gb, gm)]
    x0 = [x0h, x0g]
    for half in range(2):
        y = jnp.maximum(_mm(_r(xn[half]), _r(l1_ref[...])) +
                        _mm(_r(xc[half]), _r(l2_ref[...])) + lb_ref[...], 0.0)
        res = x0[half] * JW + y * (1.0 - JW)
        o_ref[half] = res
        if want_mean:
            aux_ref[0, half:half + 1, :] = jnp.mean(res, axis=0, keepdims=True)
        else:
            aux_ref[half] = _r(res)


def _wspec(shape):
    return pl.BlockSpec(shape, lambda g: tuple(0 for _ in shape))


_AX_SPEC = pl.BlockSpec((NC, 2, NSEG, D), lambda g: (0, g, 0, 0))
_AE_SPEC = pl.BlockSpec((NC, 2, NSEG, EW), lambda g: (0, g, 0, 0))
_X_SPEC = pl.BlockSpec((2, NSEG, D), lambda g: (g, 0, 0))


def _layer0(ax, ae, wx, we, tb, gw, gb, gm):
    return pl.pallas_call(
        _layer0_body,
        grid=(NG,),
        in_specs=[_AX_SPEC, _AE_SPEC, _wspec((D, D)), _wspec((DEE, D)),
                  _wspec((1, D)), _wspec((1, D)), _wspec((1, D)),
                  _wspec((1, D))],
        out_specs=(_X_SPEC, _X_SPEC),
        out_shape=(jax.ShapeDtypeStruct((NSEG, NSEG, D), jnp.float32),
                   jax.ShapeDtypeStruct((NSEG, NSEG, D), jnp.float32)),
    )(ax, ae, wx, we, tb, gw, gb, gm)


def _layer(x0, ax, ae, wx, we, tb, gw, gb, gm, hq, hk, hv, gq, gk, gv,
           l1, l2, lb, want_mean):
    in_specs = [_X_SPEC, _AX_SPEC, _AE_SPEC, _wspec((D, D)), _wspec((DEE, D)),
                _wspec((1, D)), _wspec((1, D)), _wspec((1, D)), _wspec((1, D))]
    in_specs += [_wspec((D, D))] * 6
    in_specs += [_wspec((D, D)), _wspec((D, D)), _wspec((1, D))]
    if want_mean:
        aux_spec = pl.BlockSpec((1, 2, D), lambda g: (g, 0, 0))
        aux_shape = jax.ShapeDtypeStruct((NG, 2, D), jnp.float32)
    else:
        aux_spec = _X_SPEC
        aux_shape = jax.ShapeDtypeStruct((NSEG, NSEG, D), jnp.float32)
    return pl.pallas_call(
        functools.partial(_layer_body, want_mean=want_mean),
        grid=(NG,),
        in_specs=in_specs,
        out_specs=(_X_SPEC, aux_spec),
        out_shape=(jax.ShapeDtypeStruct((NSEG, NSEG, D), jnp.float32),
                   aux_shape),
    )(x0, ax, ae, wx, we, tb, gw, gb, gm, hq, hk, hv, gq, gk, gv, l1, l2, lb)


def _round_body(i_ref, o_ref):
    o_ref[...] = _r(i_ref[...])


def _round_x(x):
    blk = N // 10
    return pl.pallas_call(
        _round_body,
        grid=(10,),
        in_specs=[pl.BlockSpec((blk, D), lambda i: (i, 0))],
        out_specs=pl.BlockSpec((blk, D), lambda i: (i, 0)),
        out_shape=jax.ShapeDtypeStruct((N, D), jnp.float32),
    )(x)


def _readout_body(x1_ref, x2_ref, w1_ref, b1_ref, w2_ref, b2_ref, o_ref):
    h = jnp.maximum(
        _mm(_r(x1_ref[...] - x2_ref[...]), _r(w1_ref[...])) + b1_ref[...], 0.0)
    sc = _mm(_r(h), _r(w2_ref[...])) + b2_ref[...]
    o_ref[...] = jax.nn.softmax(sc, axis=1)


def _readout(x1, x2, w1, b1, w2, b2):
    return pl.pallas_call(
        _readout_body,
        out_shape=jax.ShapeDtypeStruct((NG, 2), jnp.float32),
    )(x1, x2, w1, b1, w2, b2)


def kernel(x, edge_index, edge_features, batch, params):
    f32 = jnp.float32
    src = edge_index[0]
    dst = edge_index[1]
    npd = EPAD - E
    srcb = jnp.concatenate([src, jnp.zeros((npd,), jnp.int32)]).reshape(NW, NB, BE)
    dstb = jnp.concatenate([dst, jnp.full((npd,), N, jnp.int32)]).reshape(NW, NB, BE)

    wp = jnp.zeros((DE, EW), f32).at[:, :DEE].set(params['efm_w'])
    bp = jnp.zeros((1, EW), f32).at[0, :DEE].set(params['efm_b']).at[0, DEE].set(1.0)
    efp = jnp.concatenate([edge_features, jnp.zeros((npd, DE), f32)])
    eftp = _edge_mlp(efp, wp, bp).reshape(NW, NB, BE, EW)

    zx = jnp.zeros((NPAD, D), f32)
    ze = jnp.zeros((NPAD, EW), f32)

    gw = params['gn_w'].reshape(1, D)
    gb = params['gn_b'].reshape(1, D)
    gm = params['gn_ms'].reshape(1, D)
    l1 = params['lnm_w'][:D]
    l2 = params['lnm_w'][D:]
    lb = params['lnm_b'].reshape(1, D)

    aggE = _sc_agg_eft(eftp, srcb, dstb, ze)
    ae = aggE.reshape(NC, NSEG, NSEG, EW)
    ax = _sc_agg_x(_round_x(x), srcb, dstb, zx).reshape(NC, NSEG, NSEG, D)

    xcur = None
    xm = None
    for i in range(NLAYERS):
        wx = params['teg_w'][i, :D]
        we = params['teg_w'][i, D:]
        tb = params['teg_b'][i].reshape(1, D)
        if i == 0:
            xcur, xr = _layer0(ax, ae, wx, we, tb, gw, gb, gm)
        else:
            xcur, aux = _layer(xcur, ax, ae, wx, we, tb, gw, gb, gm,
                               params['hQ'], params['hK'], params['hV'],
                               params['gQ'], params['gK'], params['gV'],
                               l1, l2, lb, want_mean=(i == NLAYERS - 1))
            if i == NLAYERS - 1:
                xm = aux
            else:
                xr = aux
        if i < NLAYERS - 1:
            ax = _sc_agg_x(xr.reshape(N, D), srcb, dstb, zx)
            ax = ax.reshape(NC, NSEG, NSEG, D)

    xout1 = xm[:, 0]
    xout2 = xm[:, 1]
    scores = _readout(xout1, xout2, params['fl1_w'],
                      params['fl1_b'].reshape(1, D), params['fl2_w'],
                      params['fl2_b'].reshape(1, 2))
    return (scores.reshape(-1), xcur.reshape(N, D), xout1, xout2)


# speedup vs baseline: 1.0318x; 1.0318x over previous
import functools

import jax
import jax.numpy as jnp
from jax import lax
from jax.experimental import pallas as pl
from jax.experimental.pallas import tpu as pltpu
from jax.experimental.pallas import tpu_sc as plsc

N = 10000
E = 320000
D = 128
DE = 16
DEE = 32
NG = 50
NSEG = 100
NLAYERS = 4
JW = 0.5

EW = 128
NC = 2
NS = 16
NW = NC * NS
BE = 128
NB = 80
EPW = NB * BE
EPAD = NW * EPW
NPAD = 10112
ZROWS = NPAD // NS
WROWS = 632
WLAST = N - 15 * WROWS

_PREC = jax.lax.Precision.HIGHEST


def _r(t):
    return t.astype(jnp.bfloat16).astype(jnp.float32)


def _mm(a, b):
    return jax.lax.dot_general(a, b, (((a.ndim - 1,), (0,)), ((), ())),
                               precision=_PREC, preferred_element_type=jnp.float32)


def _mm_t(a, b):
    return jax.lax.dot_general(a, b, (((1,), (1,)), ((), ())),
                               precision=_PREC, preferred_element_type=jnp.float32)


def _gnorm(v, w, b, ms):
    m = jnp.mean(v, axis=0, keepdims=True)
    o = v - m * ms
    var = jnp.mean(o * o, axis=0, keepdims=True)
    return w * o / jnp.sqrt(var + 1e-5) + b


@functools.lru_cache(maxsize=None)
def _make_sc_agg(gather: bool, width: int):
    mesh = plsc.VectorSubcoreMesh(core_axis_name="c", subcore_axis_name="s",
                                  num_cores=NC, num_subcores=NS)
    out_type = jax.ShapeDtypeStruct((NC, N, width), jnp.float32)
    scratch = [
        pltpu.VMEM((NB, BE), jnp.int32),
        pltpu.VMEM((NB, BE), jnp.int32),
        pltpu.VMEM((BE, width), jnp.float32),
        pltpu.VMEM_SHARED((NPAD, width), jnp.float32),
        pltpu.SemaphoreType.DMA,
    ]

    def body(pay, srcb, dstb, z, out, srcv, dstv, buf, acc, sem):
        c = lax.axis_index("c")
        s = lax.axis_index("s")
        wid = c * NS + s

        pltpu.sync_copy(z.at[pl.ds(s * ZROWS, ZROWS)],
                        acc.at[pl.ds(s * ZROWS, ZROWS)])
        if gather:
            pltpu.sync_copy(srcb.at[wid], srcv)
        pltpu.sync_copy(dstb.at[wid], dstv)
        plsc.subcore_barrier()

        def step(j, carry):
            if gather:
                pltpu.async_copy(pay.at[srcv.at[j]], buf, sem).wait()
            else:
                pltpu.async_copy(pay.at[wid, j], buf, sem).wait()
            pltpu.sync_copy(buf, acc.at[dstv.at[j]], add=True)
            return carry

        lax.fori_loop(0, NB, step, 0)
        plsc.subcore_barrier()

        @pl.when(s < NS - 1)
        def _():
            pltpu.sync_copy(acc.at[pl.ds(s * WROWS, WROWS)],
                            out.at[c, pl.ds(s * WROWS, WROWS)])

        @pl.when(s == NS - 1)
        def _():
            pltpu.sync_copy(acc.at[pl.ds((NS - 1) * WROWS, WLAST)],
                            out.at[c, pl.ds((NS - 1) * WROWS, WLAST)])

    return pl.kernel(body, out_type=out_type, mesh=mesh, scratch_types=scratch)


def _sc_agg_x(x, srcb, dstb, zx):
    return _make_sc_agg(True, D)(x, srcb, dstb, zx)


def _sc_agg_eft(eftp, srcb, dstb, ze):
    return _make_sc_agg(False, EW)(eftp, srcb, dstb, ze)


_EBLK = 4096


def _edge_mlp_body(ef_ref, w_ref, b_ref, o_ref):
    o_ref[...] = _r(jnp.maximum(
        _mm(_r(ef_ref[...]), _r(w_ref[...])) + b_ref[...], 0.0))


def _edge_mlp(efp, wp, bp):
    return pl.pallas_call(
        _edge_mlp_body,
        grid=(EPAD // _EBLK,),
        in_specs=[
            pl.BlockSpec((_EBLK, DE), lambda i: (i, 0)),
            pl.BlockSpec((DE, EW), lambda i: (0, 0)),
            pl.BlockSpec((1, EW), lambda i: (0, 0)),
        ],
        out_specs=pl.BlockSpec((_EBLK, EW), lambda i: (i, 0)),
        out_shape=jax.ShapeDtypeStruct((EPAD, EW), jnp.float32),
    )(efp, wp, bp)


def _agg_epilogue(ax_ref, ae_ref, half, wx, we, tb, gw, gb, gm):
    agg = ax_ref[0, half] + ax_ref[1, half]
    ae = ae_ref[0, half] + ae_ref[1, half]
    cnt = ae[:, DEE:DEE + 1]
    pre = (_mm(agg, wx) + _mm(ae[:, :DEE], we) + cnt * tb) / jnp.maximum(cnt, 1.0)
    return _gnorm(jnp.maximum(pre, 0.0), gw, gb, gm)


def _layer0_body(ax_ref, ae_ref, wx_ref, we_ref, tb_ref, gw_ref, gb_ref,
                 gm_ref, o_ref, or_ref):
    wx = _r(wx_ref[...])
    we = _r(we_ref[...])
    tb = tb_ref[...]
    gw = gw_ref[...]
    gb = gb_ref[...]
    gm = gm_ref[...]
    for half in range(2):
        res = _agg_epilogue(ax_ref, ae_ref, half, wx, we, tb, gw, gb, gm)
        o_ref[half] = res
        or_ref[half] = _r(res)


def _layer_body(x0_ref, ax_ref, ae_ref, wx_ref, we_ref, tb_ref, gw_ref,
                gb_ref, gm_ref, hq_ref, hk_ref, hv_ref, gq_ref, gk_ref,
                gv_ref, l1_ref, l2_ref, lb_ref, o_ref, aux_ref, *, want_mean):
    wx = _r(wx_ref[...])
    we = _r(we_ref[...])
    tb = tb_ref[...]
    gw = gw_ref[...]
    gb = gb_ref[...]
    gm = gm_ref[...]
    xn = [_agg_epilogue(ax_ref, ae_ref, h, wx, we, tb, gw, gb, gm)
          for h in range(2)]
    x0h = x0_ref[0]
    x0g = x0_ref[1]
    x0hr = _r(x0h)
    x0gr = _r(x0g)
    hq = jnp.maximum(_mm(x0hr, _r(hq_ref[...])), 0.0)
    hk = jnp.maximum(_mm(x0hr, _r(hk_ref[...])), 0.0)
    hv = _mm(x0hr, _r(hv_ref[...]))
    gq = jnp.maximum(_mm(x0gr, _r(gq_ref[...])), 0.0)
    gk = jnp.maximum(_mm(x0gr, _r(gk_ref[...])), 0.0)
    gv = _mm(x0gr, _r(gv_ref[...]))
    ah = _mm(_r(jax.nn.softmax(_mm_t(_r(hq), _r(gk)), axis=1)), _r(gv))
    ag = _mm(_r(jax.nn.softmax(_mm_t(_r(gq), _r(hk)), axis=1)), _r(hv))
    xc = [_gnorm(ah, gw, gb, gm), _gnorm(ag, gw, gb, gm)]
    x0 = [x0h, x0g]
    for half in range(2):
        y = jnp.maximum(_mm(_r(xn[half]), _r(l1_ref[...])) +
                        _mm(_r(xc[half]), _r(l2_ref[...])) + lb_ref[...], 0.0)
        res = x0[half] * JW + y * (1.0 - JW)
        o_ref[half] = res
        if want_mean:
            aux_ref[0, half:half + 1, :] = jnp.mean(res, axis=0, keepdims=True)
        else:
            aux_ref[half] = _r(res)


def _wspec(shape):
    return pl.BlockSpec(shape, lambda g: tuple(0 for _ in shape))


_AX_SPEC = pl.BlockSpec((NC, 2, NSEG, D), lambda g: (0, g, 0, 0))
_AE_SPEC = pl.BlockSpec((NC, 2, NSEG, EW), lambda g: (0, g, 0, 0))
_X_SPEC = pl.BlockSpec((2, NSEG, D), lambda g: (g, 0, 0))


def _layer0(ax, ae, wx, we, tb, gw, gb, gm):
    return pl.pallas_call(
        _layer0_body,
        grid=(NG,),
        in_specs=[_AX_SPEC, _AE_SPEC, _wspec((D, D)), _wspec((DEE, D)),
                  _wspec((1, D)), _wspec((1, D)), _wspec((1, D)),
                  _wspec((1, D))],
        out_specs=(_X_SPEC, _X_SPEC),
        out_shape=(jax.ShapeDtypeStruct((NSEG, NSEG, D), jnp.float32),
                   jax.ShapeDtypeStruct((NSEG, NSEG, D), jnp.float32)),
    )(ax, ae, wx, we, tb, gw, gb, gm)


def _layer(x0, ax, ae, wx, we, tb, gw, gb, gm, hq, hk, hv, gq, gk, gv,
           l1, l2, lb, want_mean):
    in_specs = [_X_SPEC, _AX_SPEC, _AE_SPEC, _wspec((D, D)), _wspec((DEE, D)),
                _wspec((1, D)), _wspec((1, D)), _wspec((1, D)), _wspec((1, D))]
    in_specs += [_wspec((D, D))] * 6
    in_specs += [_wspec((D, D)), _wspec((D, D)), _wspec((1, D))]
    if want_mean:
        aux_spec = pl.BlockSpec((1, 2, D), lambda g: (g, 0, 0))
        aux_shape = jax.ShapeDtypeStruct((NG, 2, D), jnp.float32)
    else:
        aux_spec = _X_SPEC
        aux_shape = jax.ShapeDtypeStruct((NSEG, NSEG, D), jnp.float32)
    return pl.pallas_call(
        functools.partial(_layer_body, want_mean=want_mean),
        grid=(NG,),
        in_specs=in_specs,
        out_specs=(_X_SPEC, aux_spec),
        out_shape=(jax.ShapeDtypeStruct((NSEG, NSEG, D), jnp.float32),
                   aux_shape),
    )(x0, ax, ae, wx, we, tb, gw, gb, gm, hq, hk, hv, gq, gk, gv, l1, l2, lb)


def _round_body(i_ref, o_ref):
    o_ref[...] = _r(i_ref[...])


def _round_x(x):
    blk = N // 10
    return pl.pallas_call(
        _round_body,
        grid=(10,),
        in_specs=[pl.BlockSpec((blk, D), lambda i: (i, 0))],
        out_specs=pl.BlockSpec((blk, D), lambda i: (i, 0)),
        out_shape=jax.ShapeDtypeStruct((N, D), jnp.float32),
    )(x)


def _readout_body(x1_ref, x2_ref, w1_ref, b1_ref, w2_ref, b2_ref, o_ref):
    h = jnp.maximum(
        _mm(_r(x1_ref[...] - x2_ref[...]), _r(w1_ref[...])) + b1_ref[...], 0.0)
    sc = _mm(_r(h), _r(w2_ref[...])) + b2_ref[...]
    o_ref[...] = jax.nn.softmax(sc, axis=1)


def _readout(x1, x2, w1, b1, w2, b2):
    return pl.pallas_call(
        _readout_body,
        out_shape=jax.ShapeDtypeStruct((NG, 2), jnp.float32),
    )(x1, x2, w1, b1, w2, b2)


def kernel(x, edge_index, edge_features, batch, params):
    f32 = jnp.float32
    src = edge_index[0]
    dst = edge_index[1]
    npd = EPAD - E
    srcb = jnp.concatenate([src, jnp.zeros((npd,), jnp.int32)]).reshape(NW, NB, BE)
    pad_dst = N + jnp.arange(npd, dtype=jnp.int32) % (NPAD - N)
    dstb = jnp.concatenate([dst, pad_dst]).reshape(NW, NB, BE)

    wp = jnp.zeros((DE, EW), f32).at[:, :DEE].set(params['efm_w'])
    bp = jnp.zeros((1, EW), f32).at[0, :DEE].set(params['efm_b']).at[0, DEE].set(1.0)
    efp = jnp.concatenate([edge_features, jnp.zeros((npd, DE), f32)])
    eftp = _edge_mlp(efp, wp, bp).reshape(NW, NB, BE, EW)

    zx = jnp.zeros((NPAD, D), f32)
    ze = jnp.zeros((NPAD, EW), f32)

    gw = params['gn_w'].reshape(1, D)
    gb = params['gn_b'].reshape(1, D)
    gm = params['gn_ms'].reshape(1, D)
    l1 = params['lnm_w'][:D]
    l2 = params['lnm_w'][D:]
    lb = params['lnm_b'].reshape(1, D)

    aggE = _sc_agg_eft(eftp, srcb, dstb, ze)
    ae = aggE.reshape(NC, NSEG, NSEG, EW)
    ax = _sc_agg_x(_round_x(x), srcb, dstb, zx).reshape(NC, NSEG, NSEG, D)

    xcur = None
    xm = None
    for i in range(NLAYERS):
        wx = params['teg_w'][i, :D]
        we = params['teg_w'][i, D:]
        tb = params['teg_b'][i].reshape(1, D)
        if i == 0:
            xcur, xr = _layer0(ax, ae, wx, we, tb, gw, gb, gm)
        else:
            xcur, aux = _layer(xcur, ax, ae, wx, we, tb, gw, gb, gm,
                               params['hQ'], params['hK'], params['hV'],
                               params['gQ'], params['gK'], params['gV'],
                               l1, l2, lb, want_mean=(i == NLAYERS - 1))
            if i == NLAYERS - 1:
                xm = aux
            else:
                xr = aux
        if i < NLAYERS - 1:
            ax = _sc_agg_x(xr.reshape(N, D), srcb, dstb, zx)
            ax = ax.reshape(NC, NSEG, NSEG, D)

    xout1 = xm[:, 0]
    xout2 = xm[:, 1]
    scores = _readout(xout1, xout2, params['fl1_w'],
                      params['fl1_b'].reshape(1, D), params['fl2_w'],
                      params['fl2_b'].reshape(1, 2))
    return (scores.reshape(-1), xcur.reshape(N, D), xout1, xout2)


# speedup vs baseline: 1.4433x; 1.3988x over previous
import functools

import jax
import jax.numpy as jnp
from jax import lax
from jax.experimental import pallas as pl
from jax.experimental.pallas import tpu as pltpu
from jax.experimental.pallas import tpu_sc as plsc

N = 10000
E = 320000
D = 128
DE = 16
DEE = 32
NG = 50
NSEG = 100
NLAYERS = 4
JW = 0.5

EW = 128
NC = 2
NS = 16
NW = NC * NS
BE = 128
NB = 79
EPW = NB * BE
EPAD = NW * EPW
NPAD = 10112
ZROWS = NPAD // NS
WROWS = 632
WLAST = N - 15 * WROWS

_PREC = jax.lax.Precision.HIGHEST


def _r(t):
    return t.astype(jnp.bfloat16).astype(jnp.float32)


def _mm(a, b):
    return jax.lax.dot_general(a, b, (((a.ndim - 1,), (0,)), ((), ())),
                               precision=_PREC, preferred_element_type=jnp.float32)


def _mm_t(a, b):
    return jax.lax.dot_general(a, b, (((1,), (1,)), ((), ())),
                               precision=_PREC, preferred_element_type=jnp.float32)


def _gnorm(v, w, b, ms):
    m = jnp.mean(v, axis=0, keepdims=True)
    o = v - m * ms
    var = jnp.mean(o * o, axis=0, keepdims=True)
    return w * o / jnp.sqrt(var + 1e-5) + b


@functools.lru_cache(maxsize=None)
def _make_sc_agg(gather: bool, width: int):
    mesh = plsc.VectorSubcoreMesh(core_axis_name="c", subcore_axis_name="s",
                                  num_cores=NC, num_subcores=NS)
    out_type = jax.ShapeDtypeStruct((NC, N, width), jnp.float32)
    scratch = [
        pltpu.VMEM((NB, BE), jnp.int32),
        pltpu.VMEM((NB, BE), jnp.int32),
        pltpu.VMEM((BE, width), jnp.float32),
        pltpu.VMEM_SHARED((NPAD, width), jnp.float32),
        pltpu.SemaphoreType.DMA,
    ]

    def body(pay, srcb, dstb, z, out, srcv, dstv, buf, acc, sem):
        c = lax.axis_index("c")
        s = lax.axis_index("s")
        wid = c * NS + s

        pltpu.sync_copy(z.at[pl.ds(s * ZROWS, ZROWS)],
                        acc.at[pl.ds(s * ZROWS, ZROWS)])
        if gather:
            pltpu.sync_copy(srcb.at[wid], srcv)
        pltpu.sync_copy(dstb.at[wid], dstv)
        plsc.subcore_barrier()

        def step(j, carry):
            if gather:
                pltpu.async_copy(pay.at[srcv.at[j]], buf, sem).wait()
            else:
                pltpu.async_copy(pay.at[wid, j], buf, sem).wait()
            pltpu.sync_copy(buf, acc.at[dstv.at[j]], add=True)
            return carry

        lax.fori_loop(0, NB, step, 0)
        plsc.subcore_barrier()

        @pl.when(s < NS - 1)
        def _():
            pltpu.sync_copy(acc.at[pl.ds(s * WROWS, WROWS)],
                            out.at[c, pl.ds(s * WROWS, WROWS)])

        @pl.when(s == NS - 1)
        def _():
            pltpu.sync_copy(acc.at[pl.ds((NS - 1) * WROWS, WLAST)],
                            out.at[c, pl.ds((NS - 1) * WROWS, WLAST)])

    return pl.kernel(body, out_type=out_type, mesh=mesh, scratch_types=scratch)


def _sc_agg_x(x, srcb, dstb, zx):
    return _make_sc_agg(True, D)(x, srcb, dstb, zx)


def _sc_agg_eft(eftp, srcb, dstb, ze):
    return _make_sc_agg(False, EW)(eftp, srcb, dstb, ze)


_EBLK = 4096


def _edge_mlp_body(ef_ref, w_ref, b_ref, o_ref):
    o_ref[...] = _r(jnp.maximum(
        _mm(_r(ef_ref[...]), _r(w_ref[...])) + b_ref[...], 0.0))


def _edge_mlp(efp, wp, bp):
    return pl.pallas_call(
        _edge_mlp_body,
        grid=(EPAD // _EBLK,),
        in_specs=[
            pl.BlockSpec((_EBLK, DE), lambda i: (i, 0)),
            pl.BlockSpec((DE, EW), lambda i: (0, 0)),
            pl.BlockSpec((1, EW), lambda i: (0, 0)),
        ],
        out_specs=pl.BlockSpec((_EBLK, EW), lambda i: (i, 0)),
        out_shape=jax.ShapeDtypeStruct((EPAD, EW), jnp.float32),
    )(efp, wp, bp)


def _agg_epilogue(ax_ref, ae_ref, half, wx, we, tb, gw, gb, gm):
    agg = ax_ref[0, half] + ax_ref[1, half]
    ae = ae_ref[0, half] + ae_ref[1, half]
    cnt = ae[:, DEE:DEE + 1]
    pre = (_mm(agg, wx) + _mm(ae[:, :DEE], we) + cnt * tb) / jnp.maximum(cnt, 1.0)
    return _gnorm(jnp.maximum(pre, 0.0), gw, gb, gm)


def _layer0_body(ax_ref, ae_ref, wx_ref, we_ref, tb_ref, gw_ref, gb_ref,
                 gm_ref, o_ref, or_ref):
    wx = _r(wx_ref[...])
    we = _r(we_ref[...])
    tb = tb_ref[...]
    gw = gw_ref[...]
    gb = gb_ref[...]
    gm = gm_ref[...]
    for half in range(2):
        res = _agg_epilogue(ax_ref, ae_ref, half, wx, we, tb, gw, gb, gm)
        o_ref[half] = res
        or_ref[half] = _r(res)


def _layer_body(x0_ref, ax_ref, ae_ref, wx_ref, we_ref, tb_ref, gw_ref,
                gb_ref, gm_ref, hq_ref, hk_ref, hv_ref, gq_ref, gk_ref,
                gv_ref, l1_ref, l2_ref, lb_ref, o_ref, aux_ref, *, want_mean):
    wx = _r(wx_ref[...])
    we = _r(we_ref[...])
    tb = tb_ref[...]
    gw = gw_ref[...]
    gb = gb_ref[...]
    gm = gm_ref[...]
    xn = [_agg_epilogue(ax_ref, ae_ref, h, wx, we, tb, gw, gb, gm)
          for h in range(2)]
    x0h = x0_ref[0]
    x0g = x0_ref[1]
    x0hr = _r(x0h)
    x0gr = _r(x0g)
    hq = jnp.maximum(_mm(x0hr, _r(hq_ref[...])), 0.0)
    hk = jnp.maximum(_mm(x0hr, _r(hk_ref[...])), 0.0)
    hv = _mm(x0hr, _r(hv_ref[...]))
    gq = jnp.maximum(_mm(x0gr, _r(gq_ref[...])), 0.0)
    gk = jnp.maximum(_mm(x0gr, _r(gk_ref[...])), 0.0)
    gv = _mm(x0gr, _r(gv_ref[...]))
    ah = _mm(_r(jax.nn.softmax(_mm_t(_r(hq), _r(gk)), axis=1)), _r(gv))
    ag = _mm(_r(jax.nn.softmax(_mm_t(_r(gq), _r(hk)), axis=1)), _r(hv))
    xc = [_gnorm(ah, gw, gb, gm), _gnorm(ag, gw, gb, gm)]
    x0 = [x0h, x0g]
    for half in range(2):
        y = jnp.maximum(_mm(_r(xn[half]), _r(l1_ref[...])) +
                        _mm(_r(xc[half]), _r(l2_ref[...])) + lb_ref[...], 0.0)
        res = x0[half] * JW + y * (1.0 - JW)
        o_ref[half] = res
        if want_mean:
            aux_ref[0, half:half + 1, :] = jnp.mean(res, axis=0, keepdims=True)
        else:
            aux_ref[half] = _r(res)


def _wspec(shape):
    return pl.BlockSpec(shape, lambda g: tuple(0 for _ in shape))


_AX_SPEC = pl.BlockSpec((NC, 2, NSEG, D), lambda g: (0, g, 0, 0))
_AE_SPEC = pl.BlockSpec((NC, 2, NSEG, EW), lambda g: (0, g, 0, 0))
_X_SPEC = pl.BlockSpec((2, NSEG, D), lambda g: (g, 0, 0))


def _layer0(ax, ae, wx, we, tb, gw, gb, gm):
    return pl.pallas_call(
        _layer0_body,
        grid=(NG,),
        in_specs=[_AX_SPEC, _AE_SPEC, _wspec((D, D)), _wspec((DEE, D)),
                  _wspec((1, D)), _wspec((1, D)), _wspec((1, D)),
                  _wspec((1, D))],
        out_specs=(_X_SPEC, _X_SPEC),
        out_shape=(jax.ShapeDtypeStruct((NSEG, NSEG, D), jnp.float32),
                   jax.ShapeDtypeStruct((NSEG, NSEG, D), jnp.float32)),
    )(ax, ae, wx, we, tb, gw, gb, gm)


def _layer(x0, ax, ae, wx, we, tb, gw, gb, gm, hq, hk, hv, gq, gk, gv,
           l1, l2, lb, want_mean):
    in_specs = [_X_SPEC, _AX_SPEC, _AE_SPEC, _wspec((D, D)), _wspec((DEE, D)),
                _wspec((1, D)), _wspec((1, D)), _wspec((1, D)), _wspec((1, D))]
    in_specs += [_wspec((D, D))] * 6
    in_specs += [_wspec((D, D)), _wspec((D, D)), _wspec((1, D))]
    if want_mean:
        aux_spec = pl.BlockSpec((1, 2, D), lambda g: (g, 0, 0))
        aux_shape = jax.ShapeDtypeStruct((NG, 2, D), jnp.float32)
    else:
        aux_spec = _X_SPEC
        aux_shape = jax.ShapeDtypeStruct((NSEG, NSEG, D), jnp.float32)
    return pl.pallas_call(
        functools.partial(_layer_body, want_mean=want_mean),
        grid=(NG,),
        in_specs=in_specs,
        out_specs=(_X_SPEC, aux_spec),
        out_shape=(jax.ShapeDtypeStruct((NSEG, NSEG, D), jnp.float32),
                   aux_shape),
    )(x0, ax, ae, wx, we, tb, gw, gb, gm, hq, hk, hv, gq, gk, gv, l1, l2, lb)


def _round_body(i_ref, o_ref):
    o_ref[...] = _r(i_ref[...])


def _round_x(x):
    blk = N // 10
    return pl.pallas_call(
        _round_body,
        grid=(10,),
        in_specs=[pl.BlockSpec((blk, D), lambda i: (i, 0))],
        out_specs=pl.BlockSpec((blk, D), lambda i: (i, 0)),
        out_shape=jax.ShapeDtypeStruct((N, D), jnp.float32),
    )(x)


def _readout_body(x1_ref, x2_ref, w1_ref, b1_ref, w2_ref, b2_ref, o_ref):
    h = jnp.maximum(
        _mm(_r(x1_ref[...] - x2_ref[...]), _r(w1_ref[...])) + b1_ref[...], 0.0)
    sc = _mm(_r(h), _r(w2_ref[...])) + b2_ref[...]
    o_ref[...] = jax.nn.softmax(sc, axis=1)


def _readout(x1, x2, w1, b1, w2, b2):
    return pl.pallas_call(
        _readout_body,
        out_shape=jax.ShapeDtypeStruct((NG, 2), jnp.float32),
    )(x1, x2, w1, b1, w2, b2)


def kernel(x, edge_index, edge_features, batch, params):
    f32 = jnp.float32
    src = edge_index[0]
    dst = edge_index[1]
    npd = EPAD - E
    srcb = jnp.concatenate([src, jnp.zeros((npd,), jnp.int32)]).reshape(NW, NB, BE)
    pad_dst = N + jnp.arange(npd, dtype=jnp.int32) % (NPAD - N)
    dstb = jnp.concatenate([dst, pad_dst]).reshape(NW, NB, BE)

    wp = jnp.zeros((DE, EW), f32).at[:, :DEE].set(params['efm_w'])
    bp = jnp.zeros((1, EW), f32).at[0, :DEE].set(params['efm_b']).at[0, DEE].set(1.0)
    efp = jnp.concatenate([edge_features, jnp.zeros((npd, DE), f32)])
    eftp = _edge_mlp(efp, wp, bp).reshape(NW, NB, BE, EW)

    zx = jnp.zeros((NPAD, D), f32)
    ze = jnp.zeros((NPAD, EW), f32)

    gw = params['gn_w'].reshape(1, D)
    gb = params['gn_b'].reshape(1, D)
    gm = params['gn_ms'].reshape(1, D)
    l1 = params['lnm_w'][:D]
    l2 = params['lnm_w'][D:]
    lb = params['lnm_b'].reshape(1, D)

    aggE = _sc_agg_eft(eftp, srcb, dstb, ze)
    ae = aggE.reshape(NC, NSEG, NSEG, EW)
    ax = _sc_agg_x(_round_x(x), srcb, dstb, zx).reshape(NC, NSEG, NSEG, D)

    xcur = None
    xm = None
    for i in range(NLAYERS):
        wx = params['teg_w'][i, :D]
        we = params['teg_w'][i, D:]
        tb = params['teg_b'][i].reshape(1, D)
        if i == 0:
            xcur, xr = _layer0(ax, ae, wx, we, tb, gw, gb, gm)
        else:
            xcur, aux = _layer(xcur, ax, ae, wx, we, tb, gw, gb, gm,
                               params['hQ'], params['hK'], params['hV'],
                               params['gQ'], params['gK'], params['gV'],
                               l1, l2, lb, want_mean=(i == NLAYERS - 1))
            if i == NLAYERS - 1:
                xm = aux
            else:
                xr = aux
        if i < NLAYERS - 1:
            ax = _sc_agg_x(xr.reshape(N, D), srcb, dstb, zx)
            ax = ax.reshape(NC, NSEG, NSEG, D)

    xout1 = xm[:, 0]
    xout2 = xm[:, 1]
    scores = _readout(xout1, xout2, params['fl1_w'],
                      params['fl1_b'].reshape(1, D), params['fl2_w'],
                      params['fl2_b'].reshape(1, 2))
    return (scores.reshape(-1), xcur.reshape(N, D), xout1, xout2)
